# baseline (speedup 1.0000x reference)
import functools

import jax
import jax.numpy as jnp
from jax import lax
from jax.experimental import pallas as pl
from jax.experimental.pallas import tpu as pltpu
from jax.experimental.pallas import tpu_sc as plsc

X0 = 0.0
Y0 = -39.68
VOX = 0.16
GW = 432
GH = 496
HWC = GH * GW
EPS = 1e-5
NB = 2048


def _aug_block(p):
    x = p[0:1]
    y = p[1:2]
    xi = ((x - X0) / VOX).astype(jnp.int32)
    yi = ((y - Y0) / VOX).astype(jnp.int32)
    in_m = (xi >= 0) & (xi < GW) & (yi >= 0) & (yi < GH)
    w = in_m.astype(jnp.float32)
    x_c = xi.astype(jnp.float32) * VOX + X0 + VOX / 2
    y_c = yi.astype(jnp.float32) * VOX + Y0 + VOX / 2
    dx = x - x_c
    dy = y - y_c
    aug = jnp.concatenate([p[0:5], dx, dy, jnp.zeros_like(dx)], axis=0)
    idx = jnp.where(in_m, yi * GW + xi, HWC)
    return aug, w, idx


def _affine(stats_blk, g, bt):
    s = stats_blk[:, 0:1]
    q = stats_blk[:, 1:2]
    cnt = stats_blk[:, 2:3]
    m = s / cnt
    v = q / cnt - m * m
    inv = g / jnp.sqrt(v + EPS)
    return inv, bt - m * inv


def _p1_body(pts_ref, w1_ref, p1_ref, stats_out, idx_out, acc):
    i = pl.program_id(1)
    aug, w, idx = _aug_block(pts_ref[0])
    h1 = jnp.dot(w1_ref[...], aug, preferred_element_type=jnp.float32) + p1_ref[:, 0:1]

    @pl.when(i == 0)
    def _():
        acc[...] = jnp.zeros_like(acc)

    acc[:, 0:1] += jnp.sum(h1 * w, axis=1, keepdims=True)
    acc[:, 1:2] += jnp.sum(h1 * h1 * w, axis=1, keepdims=True)
    acc[:, 2:3] += jnp.sum(w) * jnp.ones((64, 1), jnp.float32)
    stats_out[...] = acc[...][None]
    idx_out[...] = idx[None]


def _p2_body(pts_ref, w1_ref, w2_ref, p1_ref, st1_ref, stats_out, acc):
    i = pl.program_id(1)
    aug, w, _ = _aug_block(pts_ref[0])
    h1 = jnp.dot(w1_ref[...], aug, preferred_element_type=jnp.float32) + p1_ref[:, 0:1]
    sc1, sh1 = _affine(st1_ref[0], p1_ref[:, 1:2], p1_ref[:, 2:3])
    a1 = jax.nn.relu(h1 * sc1 + sh1)
    h2 = jnp.dot(w2_ref[...], a1, preferred_element_type=jnp.float32) + p1_ref[:, 3:4]

    @pl.when(i == 0)
    def _():
        acc[...] = jnp.zeros_like(acc)

    acc[:, 0:1] += jnp.sum(h2 * w, axis=1, keepdims=True)
    acc[:, 1:2] += jnp.sum(h2 * h2 * w, axis=1, keepdims=True)
    acc[:, 2:3] += jnp.sum(w) * jnp.ones((64, 1), jnp.float32)
    stats_out[...] = acc[...][None]


def _p3_body(N, pts_ref, w1_ref, w2_ref, w3_ref, p1_ref, p3_ref, st1_ref,
             st2_ref, feat_out):
    i = pl.program_id(1)
    aug, _, _ = _aug_block(pts_ref[0])
    h1 = jnp.dot(w1_ref[...], aug, preferred_element_type=jnp.float32) + p1_ref[:, 0:1]
    sc1, sh1 = _affine(st1_ref[0], p1_ref[:, 1:2], p1_ref[:, 2:3])
    a1 = jax.nn.relu(h1 * sc1 + sh1)
    h2 = jnp.dot(w2_ref[...], a1, preferred_element_type=jnp.float32) + p1_ref[:, 3:4]
    sc2, sh2 = _affine(st2_ref[0], p1_ref[:, 4:5], p1_ref[:, 5:6])
    a2 = jax.nn.relu(h2 * sc2 + sh2)
    feat = jnp.dot(w3_ref[...], a2, preferred_element_type=jnp.float32) + p3_ref[:, 0:1]
    fb = feat.astype(jnp.bfloat16)
    bits = lax.bitcast_convert_type(fb, jnp.uint16)
    packed = (bits[64:128].astype(jnp.uint32) << 16) | bits[0:64].astype(jnp.uint32)
    pos = i * NB + lax.broadcasted_iota(jnp.int32, (1, NB), 1)
    packed = jnp.where(pos < N, packed, jnp.uint32(0))
    feat_out[...] = lax.bitcast_convert_type(packed, jnp.int32)[None]


def _mlp_feat(ptsT, W1, W2, W3, P1, P3, B, NP, N):
    nblk = NP // NB
    grid = (B, nblk)
    pts_spec = pl.BlockSpec((1, 8, NB), lambda b, i: (b, 0, i))
    full = lambda shape: pl.BlockSpec(shape, lambda b, i: (0,) * len(shape))
    st_spec = pl.BlockSpec((1, 64, 8), lambda b, i: (b, 0, 0))

    stats1, idx = pl.pallas_call(
        _p1_body,
        grid=grid,
        in_specs=[pts_spec, full((64, 8)), full((64, 8))],
        out_specs=[st_spec, pl.BlockSpec((1, 1, NB), lambda b, i: (b, 0, i))],
        out_shape=[
            jax.ShapeDtypeStruct((B, 64, 8), jnp.float32),
            jax.ShapeDtypeStruct((B, 1, NP), jnp.int32),
        ],
        scratch_shapes=[pltpu.VMEM((64, 8), jnp.float32)],
        compiler_params=pltpu.CompilerParams(
            dimension_semantics=("arbitrary", "arbitrary")),
    )(ptsT, W1, P1)

    stats2 = pl.pallas_call(
        _p2_body,
        grid=grid,
        in_specs=[pts_spec, full((64, 8)), full((64, 64)), full((64, 8)), st_spec],
        out_specs=st_spec,
        out_shape=jax.ShapeDtypeStruct((B, 64, 8), jnp.float32),
        scratch_shapes=[pltpu.VMEM((64, 8), jnp.float32)],
        compiler_params=pltpu.CompilerParams(
            dimension_semantics=("arbitrary", "arbitrary")),
    )(ptsT, W1, W2, P1, stats1)

    featP = pl.pallas_call(
        functools.partial(_p3_body, N),
        grid=grid,
        in_specs=[pts_spec, full((64, 8)), full((64, 64)), full((128, 64)),
                  full((64, 8)), full((128, 8)), st_spec, st_spec],
        out_specs=pl.BlockSpec((1, 64, NB), lambda b, i: (b, 0, i)),
        out_shape=jax.ShapeDtypeStruct((B, 64, NP), jnp.int32),
        compiler_params=pltpu.CompilerParams(
            dimension_semantics=("arbitrary", "arbitrary")),
    )(ptsT, W1, W2, W3, P1, P3, stats1, stats2)

    return featP, idx


NC, NS = 2, 16
NWORK = NC * NS
RPW = 6704
WPAD = NWORK * RPW
SCH = 1024
CC = 8 * GW


def _winner_body(B, NP, N, idx_hbm, win_hbm, ibuf0, ibuf1, win_buf, sem0, sem1):
    wid = lax.axis_index("s") * NC + lax.axis_index("c")
    lo = wid * RPW
    lane = lax.iota(jnp.int32, 16)
    empty = jnp.full((16,), N, jnp.int32)
    NCH = NP // SCH
    bufs = ((ibuf0, sem0), (ibuf1, sem1))

    for b in range(B):
        def init_body(i, _):
            win_buf[pl.ds(i * 16, 16)] = empty
            return 0
        lax.fori_loop(0, RPW // 16, init_body, 0, unroll=8)

        base = b * NP
        pltpu.async_copy(idx_hbm.at[pl.ds(base, SCH)], ibuf0, sem0)
        pltpu.async_copy(idx_hbm.at[pl.ds(base + SCH, SCH)], ibuf1, sem1)

        def pair_body(g, _):
            for par, (buf, sem) in enumerate(bufs):
                ci = 2 * g + par
                pltpu.make_async_copy(
                    idx_hbm.at[pl.ds(base, SCH)], buf, sem).wait()

                def j_body(j, _):
                    cs = [buf[pl.ds((j * 8 + k) * 16, 16)] for k in range(8)]
                    lasts = [plsc.scan_count(c)[1] for c in cs]
                    for k in range(8):
                        cells = cs[k]
                        n = (ci * SCH + (j * 8 + k) * 16) + lane
                        off = cells - lo
                        m = lasts[k] & (off.astype(jnp.uint32)
                                        < jnp.uint32(RPW))
                        plsc.store_scatter(win_buf, [off], n, mask=m)
                    return 0

                lax.fori_loop(0, SCH // 128, j_body, 0)

                @pl.when(ci + 2 < NCH)
                def _():
                    pltpu.async_copy(
                        idx_hbm.at[pl.ds(base + (ci + 2) * SCH, SCH)],
                        buf, sem)
            return 0

        lax.fori_loop(0, NCH // 2, pair_body, 0)
        pltpu.sync_copy(win_buf, win_hbm.at[pl.ds(b * WPAD + lo, RPW)])


def _gather_body(B, NP, N, feat_hbm, win_hbm, out_hbm, plane_buf,
                 wbuf, olo, ohi, lsem, hsem):
    wid = lax.axis_index("s") * NC + lax.axis_index("c")
    NCC = GH // 8

    for b in range(B):
        wbase = b * WPAD
        for p in range(2):
            row = wid * 2 + p
            pltpu.sync_copy(feat_hbm.at[pl.ds((b * 64 + row) * NP, N + 16)],
                            plane_buf)

            def chunk_body(rg, _):
                @pl.when(rg >= 1)
                def _():
                    pltpu.make_async_copy(
                        olo, out_hbm.at[row, pl.ds(0, 8)], lsem).wait()
                    pltpu.make_async_copy(
                        ohi, out_hbm.at[row + 64, pl.ds(0, 8)], hsem).wait()

                for half in range(2):
                    pltpu.sync_copy(
                        win_hbm.at[pl.ds(wbase + rg * CC + half * (CC // 2),
                                         CC // 2)], wbuf)

                    def u_body(u, _):
                        ws = [wbuf[pl.ds(r * GW + u * 16, 16)]
                              for r in range(4)]
                        gs = [plsc.load_gather(plane_buf, [w]) for w in ws]
                        for r in range(4):
                            orow = half * 4 + r
                            olo[orow, pl.ds(u * 16, 16)] = plsc.bitcast(
                                gs[r] << 16, jnp.float32)
                            ohi[orow, pl.ds(u * 16, 16)] = plsc.bitcast(
                                gs[r] & jnp.int32(-65536), jnp.float32)
                        return 0

                    lax.fori_loop(0, GW // 16, u_body, 0)
                pltpu.async_copy(olo, out_hbm.at[row, pl.ds(rg * 8, 8)], lsem)
                pltpu.async_copy(ohi, out_hbm.at[row + 64, pl.ds(rg * 8, 8)],
                                 hsem)
                return 0

            lax.fori_loop(0, NCC, chunk_body, 0)
            pltpu.make_async_copy(
                olo, out_hbm.at[row, pl.ds(0, 8)], lsem).wait()
            pltpu.make_async_copy(
                ohi, out_hbm.at[row + 64, pl.ds(0, 8)], hsem).wait()


def _sc_scatter(featP, idx, NP, N):
    mesh = plsc.VectorSubcoreMesh(core_axis_name="c", subcore_axis_name="s",
                                  num_cores=NC, num_subcores=NS)
    winner = pl.kernel(
        functools.partial(_winner_body, 1, NP, N),
        out_type=jax.ShapeDtypeStruct((WPAD,), jnp.int32),
        mesh=mesh,
        scratch_types=[
            pltpu.VMEM((SCH,), jnp.int32),
            pltpu.VMEM((SCH,), jnp.int32),
            pltpu.VMEM((RPW,), jnp.int32),
            pltpu.SemaphoreType.DMA,
            pltpu.SemaphoreType.DMA,
        ],
        compiler_params=pltpu.CompilerParams(needs_layout_passes=False),
    )(idx)

    canvas = pl.kernel(
        functools.partial(_gather_body, 1, NP, N),
        out_type=jax.ShapeDtypeStruct((128, GH, GW), jnp.float32),
        mesh=mesh,
        scratch_types=[
            pltpu.VMEM((N + 16,), jnp.int32),
            pltpu.VMEM((CC // 2,), jnp.int32),
            pltpu.VMEM((8, GW), jnp.float32),
            pltpu.VMEM((8, GW), jnp.float32),
            pltpu.SemaphoreType.DMA,
            pltpu.SemaphoreType.DMA,
        ],
        compiler_params=pltpu.CompilerParams(needs_layout_passes=False),
    )(featP, winner)
    return canvas


def kernel(points_list, W1, b1, g1, bt1, W2, b2, g2, bt2, W3, b3):
    B, N, _ = points_list.shape
    NP = ((N + NB - 1) // NB) * NB
    if NP == N:
        NP += NB
    pts = jnp.pad(points_list, ((0, 0), (0, NP - N), (0, 0)),
                  constant_values=-1e4)
    ptsT = jnp.swapaxes(pts, 1, 2)
    ptsT = jnp.pad(ptsT, ((0, 0), (0, 3), (0, 0)))
    P1 = jnp.stack([b1, g1, bt1, b2, g2, bt2, b1, b1], axis=1)
    P3 = jnp.pad(b3[:, None], ((0, 0), (0, 7)))

    feats = [_mlp_feat(ptsT[b:b + 1], W1, W2, W3, P1, P3, 1, NP, N)
             for b in range(B)]
    outs = []
    for b in range(B):
        featP, idx = feats[b]
        outs.append(_sc_scatter(featP.reshape(-1), idx.reshape(-1), NP, N))
    return jnp.stack(outs)


# speedup vs baseline: 1.1186x; 1.1186x over previous
import functools

import jax
import jax.numpy as jnp
from jax import lax
from jax.experimental import pallas as pl
from jax.experimental.pallas import tpu as pltpu
from jax.experimental.pallas import tpu_sc as plsc

X0 = 0.0
Y0 = -39.68
VOX = 0.16
GW = 432
GH = 496
HWC = GH * GW
EPS = 1e-5
NB = 2048


def _aug_block(p):
    x = p[0:1]
    y = p[1:2]
    xi = ((x - X0) / VOX).astype(jnp.int32)
    yi = ((y - Y0) / VOX).astype(jnp.int32)
    in_m = (xi >= 0) & (xi < GW) & (yi >= 0) & (yi < GH)
    w = in_m.astype(jnp.float32)
    x_c = xi.astype(jnp.float32) * VOX + X0 + VOX / 2
    y_c = yi.astype(jnp.float32) * VOX + Y0 + VOX / 2
    dx = x - x_c
    dy = y - y_c
    aug = jnp.concatenate([p[0:5], dx, dy, jnp.zeros_like(dx)], axis=0)
    idx = jnp.where(in_m, yi * GW + xi, HWC)
    return aug, w, idx


def _affine(stats_blk, g, bt):
    s = stats_blk[:, 0:1]
    q = stats_blk[:, 1:2]
    cnt = stats_blk[:, 2:3]
    m = s / cnt
    v = q / cnt - m * m
    inv = g / jnp.sqrt(v + EPS)
    return inv, bt - m * inv


def _p1_body(pts_ref, w1_ref, p1_ref, stats_out, idx_out, acc):
    i = pl.program_id(1)
    aug, w, idx = _aug_block(pts_ref[0])
    h1 = jnp.dot(w1_ref[...], aug, preferred_element_type=jnp.float32) + p1_ref[:, 0:1]

    @pl.when(i == 0)
    def _():
        acc[...] = jnp.zeros_like(acc)

    acc[:, 0:1] += jnp.sum(h1 * w, axis=1, keepdims=True)
    acc[:, 1:2] += jnp.sum(h1 * h1 * w, axis=1, keepdims=True)
    acc[:, 2:3] += jnp.sum(w) * jnp.ones((64, 1), jnp.float32)
    stats_out[...] = acc[...][None]
    idx_out[...] = idx[None]


def _p2_body(pts_ref, w1_ref, w2_ref, p1_ref, st1_ref, stats_out, acc):
    i = pl.program_id(1)
    aug, w, _ = _aug_block(pts_ref[0])
    h1 = jnp.dot(w1_ref[...], aug, preferred_element_type=jnp.float32) + p1_ref[:, 0:1]
    sc1, sh1 = _affine(st1_ref[0], p1_ref[:, 1:2], p1_ref[:, 2:3])
    a1 = jax.nn.relu(h1 * sc1 + sh1)
    h2 = jnp.dot(w2_ref[...], a1, preferred_element_type=jnp.float32) + p1_ref[:, 3:4]

    @pl.when(i == 0)
    def _():
        acc[...] = jnp.zeros_like(acc)

    acc[:, 0:1] += jnp.sum(h2 * w, axis=1, keepdims=True)
    acc[:, 1:2] += jnp.sum(h2 * h2 * w, axis=1, keepdims=True)
    acc[:, 2:3] += jnp.sum(w) * jnp.ones((64, 1), jnp.float32)
    stats_out[...] = acc[...][None]


def _p3_body(N, pts_ref, w1_ref, w2_ref, w3_ref, p1_ref, p3_ref, st1_ref,
             st2_ref, feat_out):
    i = pl.program_id(1)
    aug, _, _ = _aug_block(pts_ref[0])
    h1 = jnp.dot(w1_ref[...], aug, preferred_element_type=jnp.float32) + p1_ref[:, 0:1]
    sc1, sh1 = _affine(st1_ref[0], p1_ref[:, 1:2], p1_ref[:, 2:3])
    a1 = jax.nn.relu(h1 * sc1 + sh1)
    h2 = jnp.dot(w2_ref[...], a1, preferred_element_type=jnp.float32) + p1_ref[:, 3:4]
    sc2, sh2 = _affine(st2_ref[0], p1_ref[:, 4:5], p1_ref[:, 5:6])
    a2 = jax.nn.relu(h2 * sc2 + sh2)
    feat = jnp.dot(w3_ref[...], a2, preferred_element_type=jnp.float32) + p3_ref[:, 0:1]
    fb = feat.astype(jnp.bfloat16)
    bits = lax.bitcast_convert_type(fb, jnp.uint16)
    packed = (bits[64:128].astype(jnp.uint32) << 16) | bits[0:64].astype(jnp.uint32)
    pos = i * NB + lax.broadcasted_iota(jnp.int32, (1, NB), 1)
    packed = jnp.where(pos < N, packed, jnp.uint32(0))
    feat_out[...] = lax.bitcast_convert_type(packed, jnp.int32)[None]


def _mlp_feat(ptsT, W1, W2, W3, P1, P3, B, NP, N):
    nblk = NP // NB
    grid = (B, nblk)
    pts_spec = pl.BlockSpec((1, 8, NB), lambda b, i: (b, 0, i))
    full = lambda shape: pl.BlockSpec(shape, lambda b, i: (0,) * len(shape))
    st_spec = pl.BlockSpec((1, 64, 8), lambda b, i: (b, 0, 0))

    stats1, idx = pl.pallas_call(
        _p1_body,
        grid=grid,
        in_specs=[pts_spec, full((64, 8)), full((64, 8))],
        out_specs=[st_spec, pl.BlockSpec((1, 1, NB), lambda b, i: (b, 0, i))],
        out_shape=[
            jax.ShapeDtypeStruct((B, 64, 8), jnp.float32),
            jax.ShapeDtypeStruct((B, 1, NP), jnp.int32),
        ],
        scratch_shapes=[pltpu.VMEM((64, 8), jnp.float32)],
        compiler_params=pltpu.CompilerParams(
            dimension_semantics=("arbitrary", "arbitrary")),
    )(ptsT, W1, P1)

    stats2 = pl.pallas_call(
        _p2_body,
        grid=grid,
        in_specs=[pts_spec, full((64, 8)), full((64, 64)), full((64, 8)), st_spec],
        out_specs=st_spec,
        out_shape=jax.ShapeDtypeStruct((B, 64, 8), jnp.float32),
        scratch_shapes=[pltpu.VMEM((64, 8), jnp.float32)],
        compiler_params=pltpu.CompilerParams(
            dimension_semantics=("arbitrary", "arbitrary")),
    )(ptsT, W1, W2, P1, stats1)

    featP = pl.pallas_call(
        functools.partial(_p3_body, N),
        grid=grid,
        in_specs=[pts_spec, full((64, 8)), full((64, 64)), full((128, 64)),
                  full((64, 8)), full((128, 8)), st_spec, st_spec],
        out_specs=pl.BlockSpec((1, 64, NB), lambda b, i: (b, 0, i)),
        out_shape=jax.ShapeDtypeStruct((B, 64, NP), jnp.int32),
        compiler_params=pltpu.CompilerParams(
            dimension_semantics=("arbitrary", "arbitrary")),
    )(ptsT, W1, W2, W3, P1, P3, stats1, stats2)

    return featP, idx


NC, NS = 2, 16
NWORK = NC * NS
RPW = 6704
WPAD = NWORK * RPW
SCH = 1024
CC = 1728


def _winner_body(B, NP, N, idx_hbm, win_hbm, ibuf0, ibuf1, win_buf, sem0,
                 sem1):
    wid = lax.axis_index("s") * NC + lax.axis_index("c")
    lo = wid * RPW
    lane = lax.iota(jnp.int32, 16)
    empty = jnp.full((16,), N, jnp.int32)
    NCH = NP // SCH
    bufs = ((ibuf0, sem0), (ibuf1, sem1))

    for b in range(B):
        def init_body(i, _):
            win_buf[pl.ds(i * 16, 16)] = empty
            return 0
        lax.fori_loop(0, RPW // 16, init_body, 0, unroll=8)

        base = b * NP
        pltpu.async_copy(idx_hbm.at[pl.ds(base, SCH)], ibuf0, sem0)
        pltpu.async_copy(idx_hbm.at[pl.ds(base + SCH, SCH)], ibuf1, sem1)

        def pair_body(g, _):
            for par, (buf, sem) in enumerate(bufs):
                ci = 2 * g + par
                pltpu.make_async_copy(
                    idx_hbm.at[pl.ds(base, SCH)], buf, sem).wait()

                def j_body(j, _):
                    cs = [buf[pl.ds((j * 8 + k) * 16, 16)] for k in range(8)]
                    lasts = [plsc.scan_count(c)[1] for c in cs]
                    for k in range(8):
                        cells = cs[k]
                        n = (ci * SCH + (j * 8 + k) * 16) + lane
                        off = cells - lo
                        m = lasts[k] & (off.astype(jnp.uint32)
                                        < jnp.uint32(RPW))
                        plsc.store_scatter(win_buf, [off], n, mask=m)
                    return 0

                lax.fori_loop(0, SCH // 128, j_body, 0)

                @pl.when(ci + 2 < NCH)
                def _():
                    pltpu.async_copy(
                        idx_hbm.at[pl.ds(base + (ci + 2) * SCH, SCH)],
                        buf, sem)
            return 0

        lax.fori_loop(0, NCH // 2, pair_body, 0)
        pltpu.sync_copy(win_buf, win_hbm.at[pl.ds(b * WPAD + lo, RPW)])


def _gather_groups(plane_buf, wbuf, olo, ohi, offs):
    ws = [wbuf[pl.ds(o, 16)] for o in offs]
    gs = [plsc.load_gather(plane_buf, [w]) for w in ws]
    for o, g in zip(offs, gs):
        olo[pl.ds(o, 16)] = plsc.bitcast(g << 16, jnp.float32)
        ohi[pl.ds(o, 16)] = plsc.bitcast(g & jnp.int32(-65536), jnp.float32)


def _gather_body(B, NP, N, feat_hbm, win_hbm, out_hbm, plane_buf,
                 wbuf0, wbuf1, olo0, olo1, ohi0, ohi1,
                 wsem0, wsem1, lsem0, lsem1, hsem0, hsem1):
    wid = lax.axis_index("s") * NC + lax.axis_index("c")
    NCC = HWC // CC
    bufs = ((wbuf0, wsem0, olo0, lsem0, ohi0, hsem0),
            (wbuf1, wsem1, olo1, lsem1, ohi1, hsem1))

    for b in range(B):
        wbase = b * WPAD
        for p in range(2):
            row = wid * 2 + p
            PL = min(N + 16, NP)
            pltpu.sync_copy(feat_hbm.at[pl.ds((b * 64 + row) * NP, PL)],
                            plane_buf.at[pl.ds(0, PL)])
            lobase = (b * 128 + row) * HWC
            hibase = (b * 128 + row + 64) * HWC

            pltpu.async_copy(win_hbm.at[pl.ds(wbase, CC)], wbuf0, wsem0)
            pltpu.async_copy(win_hbm.at[pl.ds(wbase + CC, CC)], wbuf1, wsem1)

            def pair_body(g, _):
                for par, (wbuf, wsem, olo, lsem, ohi, hsem) in enumerate(bufs):
                    ci = 2 * g + par
                    pltpu.make_async_copy(
                        win_hbm.at[pl.ds(wbase, CC)], wbuf, wsem).wait()

                    @pl.when(ci >= 2)
                    def _():
                        pltpu.make_async_copy(
                            olo, out_hbm.at[pl.ds(lobase, CC)], lsem).wait()
                        pltpu.make_async_copy(
                            ohi, out_hbm.at[pl.ds(hibase, CC)], hsem).wait()

                    def j_batch(j, _):
                        base = j * 144
                        _gather_groups(
                            plane_buf, wbuf, olo, ohi,
                            [base + k * 16 for k in range(9)])
                        return 0

                    lax.fori_loop(0, 12, j_batch, 0)

                    pltpu.async_copy(
                        olo, out_hbm.at[pl.ds(lobase + ci * CC, CC)], lsem)
                    pltpu.async_copy(
                        ohi, out_hbm.at[pl.ds(hibase + ci * CC, CC)], hsem)

                    @pl.when(ci + 2 < NCC)
                    def _():
                        pltpu.async_copy(
                            win_hbm.at[pl.ds(wbase + (ci + 2) * CC, CC)],
                            wbuf, wsem)
                return 0

            lax.fori_loop(0, NCC // 2, pair_body, 0)
            for buf, base_, sem in ((olo0, lobase, lsem0), (olo1, lobase, lsem1),
                                    (ohi0, hibase, hsem0), (ohi1, hibase, hsem1)):
                pltpu.make_async_copy(
                    buf, out_hbm.at[pl.ds(base_, CC)], sem).wait()


def _sc_scatter(featP, idx, NP, N):
    mesh = plsc.VectorSubcoreMesh(core_axis_name="c", subcore_axis_name="s",
                                  num_cores=NC, num_subcores=NS)
    winner = pl.kernel(
        functools.partial(_winner_body, 1, NP, N),
        out_type=jax.ShapeDtypeStruct((WPAD,), jnp.int32),
        mesh=mesh,
        scratch_types=[
            pltpu.VMEM((SCH,), jnp.int32),
            pltpu.VMEM((SCH,), jnp.int32),
            pltpu.VMEM((RPW,), jnp.int32),
            pltpu.SemaphoreType.DMA,
            pltpu.SemaphoreType.DMA,
        ],
        compiler_params=pltpu.CompilerParams(needs_layout_passes=False),
    )(idx)

    canvas = pl.kernel(
        functools.partial(_gather_body, 1, NP, N),
        out_type=jax.ShapeDtypeStruct((128 * HWC,), jnp.float32),
        mesh=mesh,
        scratch_types=[
            pltpu.VMEM((N + 16,), jnp.int32),
            pltpu.VMEM((CC,), jnp.int32),
            pltpu.VMEM((CC,), jnp.int32),
            pltpu.VMEM((CC,), jnp.float32),
            pltpu.VMEM((CC,), jnp.float32),
            pltpu.VMEM((CC,), jnp.float32),
            pltpu.VMEM((CC,), jnp.float32),
            pltpu.SemaphoreType.DMA,
            pltpu.SemaphoreType.DMA,
            pltpu.SemaphoreType.DMA,
            pltpu.SemaphoreType.DMA,
            pltpu.SemaphoreType.DMA,
            pltpu.SemaphoreType.DMA,
        ],
        compiler_params=pltpu.CompilerParams(needs_layout_passes=False),
    )(featP, winner)
    return canvas


def kernel(points_list, W1, b1, g1, bt1, W2, b2, g2, bt2, W3, b3):
    B, N, _ = points_list.shape
    NP = ((N + NB - 1) // NB) * NB
    if NP == N:
        NP += NB
    pts = jnp.pad(points_list, ((0, 0), (0, NP - N), (0, 0)),
                  constant_values=-1e4)
    ptsT = jnp.swapaxes(pts, 1, 2)
    ptsT = jnp.pad(ptsT, ((0, 0), (0, 3), (0, 0)))
    P1 = jnp.stack([b1, g1, bt1, b2, g2, bt2, b1, b1], axis=1)
    P3 = jnp.pad(b3[:, None], ((0, 0), (0, 7)))

    feats = [_mlp_feat(ptsT[b:b + 1], W1, W2, W3, P1, P3, 1, NP, N)
             for b in range(B)]
    outs = []
    for b in range(B):
        featP, idx = feats[b]
        canvas = _sc_scatter(featP.reshape(-1), idx.reshape(-1), NP, N)
        outs.append(canvas.reshape(128, GH, GW))
    return jnp.stack(outs)
